# Initial kernel scaffold; baseline (speedup 1.0000x reference)
import jax
import jax.numpy as jnp
from jax.experimental import pallas as pl


def kernel(history, us, edge_index, control_idx, W_enc, b_enc, W_self, W_nbr, W_u, b_h, W_dec, b_dec):
    raise NotImplementedError("write your pallas kernel here")


# speedup vs baseline: 2.6232x; 2.6232x over previous
import jax
import jax.numpy as jnp
from jax import lax
from jax.experimental import pallas as pl
from jax.experimental.pallas import tpu as pltpu
from jax.experimental.pallas import tpu_sc as plsc

NN = 10000
BSZ = 2
HID = 256
HALF = 128
CH = 128
NTILE = 16
OWN = NN // NTILE
TILE_ROWS = 640
ACC_ROWS = NTILE * TILE_ROWS


def _sc_mesh():
    return plsc.VectorSubcoreMesh(core_axis_name="c", subcore_axis_name="s")


def _seg_sum_body(h_hbm, srcp_hbm, dstp_hbm, meta_hbm, zer_hbm, msg_hbm,
                  idx_v, dst_v, rows_v, m16, acc_sh, sem):
    cid = lax.axis_index("c")
    sid = lax.axis_index("s")

    mbase = pl.multiple_of(sid * 8, 8)
    pltpu.sync_copy(meta_hbm.at[pl.ds(mbase, NTILE)], m16)
    nch = m16[...][0]

    for g in range(BSZ):
        pltpu.sync_copy(zer_hbm, acc_sh.at[pl.ds(sid * TILE_ROWS, TILE_ROWS)])
        plsc.subcore_barrier()

        def body(k, carry):
            eb = pl.multiple_of((k * NTILE + sid) * CH, CH)
            pltpu.sync_copy(srcp_hbm.at[cid, pl.ds(eb, CH)], idx_v)
            pltpu.sync_copy(dstp_hbm.at[pl.ds(eb, CH)], dst_v)
            if g:
                for kk in range(CH // 16):
                    idx_v[pl.ds(kk * 16, 16)] = idx_v[pl.ds(kk * 16, 16)] + g * NN
            pltpu.async_copy(h_hbm.at[idx_v], rows_v, sem).wait()
            pltpu.sync_copy(rows_v, acc_sh.at[dst_v], add=True)
            return carry

        lax.fori_loop(0, nch, body, 0)
        plsc.subcore_barrier()
        out_base = cid * (BSZ * NN) + g * NN + sid * TILE_ROWS

        @pl.when(sid < NTILE - 1)
        def _():
            pltpu.sync_copy(acc_sh.at[pl.ds(sid * TILE_ROWS, TILE_ROWS)],
                            msg_hbm.at[pl.ds(out_base, TILE_ROWS)])

        @pl.when(sid == NTILE - 1)
        def _():
            nlast = NN - (NTILE - 1) * TILE_ROWS
            pltpu.sync_copy(acc_sh.at[pl.ds((NTILE - 1) * TILE_ROWS, nlast)],
                            msg_hbm.at[pl.ds(out_base, nlast)])

        plsc.subcore_barrier()


def _seg_sum(h_flat, srcp, dstp, meta, zer):
    fn = pl.kernel(
        _seg_sum_body,
        out_type=jax.ShapeDtypeStruct((2 * BSZ * NN, HALF), jnp.float32),
        mesh=_sc_mesh(),
        scratch_types=[
            pltpu.VMEM((CH,), jnp.int32),
            pltpu.VMEM((CH,), jnp.int32),
            pltpu.VMEM((CH, HALF), jnp.float32),
            pltpu.VMEM((16,), jnp.int32),
            pltpu.VMEM_SHARED((ACC_ROWS, HALF), jnp.float32),
            pltpu.SemaphoreType.DMA,
        ],
    )
    return fn(h_flat, srcp, dstp, meta, zer)


ROWS_BLK = 1000
GRID = (BSZ * NN) // ROWS_BLK


def _enc_body(hist_ref, we_ref, be_ref, out_ref):
    res = jnp.dot(hist_ref[...], we_ref[...], preferred_element_type=jnp.float32)
    res = jnp.maximum(res + be_ref[...], 0.0)
    out_ref[0] = res[:, :HALF]
    out_ref[1] = res[:, HALF:]


def _encode(history, W_enc, b_enc):
    return pl.pallas_call(
        _enc_body,
        grid=(GRID,),
        in_specs=[
            pl.BlockSpec((ROWS_BLK, history.shape[1]), lambda i: (i, 0)),
            pl.BlockSpec(W_enc.shape, lambda i: (0, 0)),
            pl.BlockSpec((1, HID), lambda i: (0, 0)),
        ],
        out_specs=pl.BlockSpec((2, ROWS_BLK, HALF), lambda i: (0, i, 0)),
        out_shape=jax.ShapeDtypeStruct((2, BSZ * NN, HALF), jnp.float32),
        compiler_params=pltpu.CompilerParams(
            dimension_semantics=("arbitrary",)),
    )(history, W_enc, b_enc)


def _upd_body(h_ref, m_ref, u_ref, x_ref, ws_ref, wn_ref, wu_ref, bh_ref,
              wd_ref, bd_ref, hout_ref, xout_ref, psq_ref):
    h = jnp.concatenate([h_ref[0], h_ref[1]], axis=1)
    m = jnp.concatenate([m_ref[0], m_ref[1]], axis=1)
    acc = jnp.dot(h, ws_ref[...], preferred_element_type=jnp.float32)
    acc += jnp.dot(m, wn_ref[...], preferred_element_type=jnp.float32)
    acc += u_ref[...] * wu_ref[...]
    acc += bh_ref[...]
    hn = jnp.maximum(acc, 0.0)
    hout_ref[0] = hn[:, :HALF]
    hout_ref[1] = hn[:, HALF:]
    xcol = jnp.dot(hn, wd_ref[...], preferred_element_type=jnp.float32)
    xout_ref[...] = x_ref[...] + xcol + bd_ref[...]
    par = lax.broadcasted_iota(jnp.int32, (ROWS_BLK, 1), 0) % 2
    sq = hn * hn
    even = jnp.sum(jnp.where(par == 0, sq, 0.0))
    odd = jnp.sum(jnp.where(par == 1, sq, 0.0))
    r = lax.broadcasted_iota(jnp.int32, (8, 128), 0)
    c = lax.broadcasted_iota(jnp.int32, (8, 128), 1)
    blk = jnp.where((r == 0) & (c == 0), even,
                    jnp.where((r == 0) & (c == 1), odd, 0.0))
    psq_ref[...] = blk[None]


def _update(h_hal, msg_hal, u_col, x_col, W_self, W_nbr, W_u, b_h2, W_dec, b_d2):
    return pl.pallas_call(
        _upd_body,
        grid=(GRID,),
        in_specs=[
            pl.BlockSpec((2, ROWS_BLK, HALF), lambda i: (0, i, 0)),
            pl.BlockSpec((2, ROWS_BLK, HALF), lambda i: (0, i, 0)),
            pl.BlockSpec((ROWS_BLK, 1), lambda i: (i, 0)),
            pl.BlockSpec((ROWS_BLK, 1), lambda i: (i, 0)),
            pl.BlockSpec((HID, HID), lambda i: (0, 0)),
            pl.BlockSpec((HID, HID), lambda i: (0, 0)),
            pl.BlockSpec((1, HID), lambda i: (0, 0)),
            pl.BlockSpec((1, HID), lambda i: (0, 0)),
            pl.BlockSpec((HID, 1), lambda i: (0, 0)),
            pl.BlockSpec((1, 1), lambda i: (0, 0)),
        ],
        out_specs=[
            pl.BlockSpec((2, ROWS_BLK, HALF), lambda i: (0, i, 0)),
            pl.BlockSpec((ROWS_BLK, 1), lambda i: (i, 0)),
            pl.BlockSpec((1, 8, 128), lambda i: (i, 0, 0)),
        ],
        out_shape=[
            jax.ShapeDtypeStruct((2, BSZ * NN, HALF), jnp.float32),
            jax.ShapeDtypeStruct((BSZ * NN, 1), jnp.float32),
            jax.ShapeDtypeStruct((GRID, 8, 128), jnp.float32),
        ],
        compiler_params=pltpu.CompilerParams(
            dimension_semantics=("arbitrary",)),
    )(h_hal, msg_hal, u_col, x_col, W_self, W_nbr, W_u, b_h2, W_dec, b_d2)


def kernel(history, us, edge_index, control_idx, W_enc, b_enc, W_self, W_nbr,
           W_u, b_h, W_dec, b_dec):
    T = us.shape[1]
    E = edge_index.shape[1]
    src = edge_index[0]
    dst = edge_index[1]

    order = jnp.argsort(dst, stable=True)
    src_s = src[order]
    dst_s = dst[order]
    t_e = dst_s // OWN
    cnt = jnp.bincount(t_e, length=NTILE)
    nch = (cnt + CH - 1) // CH
    start = jnp.cumsum(cnt) - cnt
    rank = jnp.arange(E, dtype=jnp.int32) - start[t_e]
    pos = ((rank // CH) * NTILE + t_e) * CH + rank % CH
    nchmax = (E + CH - 1) // CH
    epadb = NTILE * nchmax * CH
    srcp = jnp.zeros((epadb,), jnp.int32).at[pos].set(src_s)
    dstp = jnp.full((epadb,), NN, jnp.int32).at[pos].set(dst_s)
    srcp = srcp[None, :] + (jnp.arange(2, dtype=jnp.int32) * BSZ * NN)[:, None]
    meta = jnp.zeros((NTILE * 8,), jnp.int32).at[
        jnp.arange(NTILE) * 8].set(nch.astype(jnp.int32))
    zer = jnp.zeros((TILE_ROWS, HALF), jnp.float32)

    rows = jnp.concatenate([control_idx + g * NN for g in range(BSZ)])
    u_all = jnp.zeros((T, BSZ * NN), jnp.float32).at[:, rows].set(us[:, :, 0].T)

    b_e2 = b_enc.reshape(1, HID)
    b_h2 = b_h.reshape(1, HID)
    b_d2 = b_dec.reshape(1, 1)

    h_hal = _encode(history, W_enc, b_e2)
    x_col = history[:, -1:]

    xs_cols = []
    psq_first = psq_last = None
    for t in range(T):
        msg_flat = _seg_sum(h_hal.reshape(2 * BSZ * NN, HALF),
                            srcp, dstp, meta, zer)
        h_hal, x_col, psq = _update(h_hal, msg_flat.reshape(2, BSZ * NN, HALF),
                                    u_all[t].reshape(BSZ * NN, 1), x_col,
                                    W_self, W_nbr, W_u, b_h2, W_dec, b_d2)
        xs_cols.append(x_col)
        if t == 0:
            psq_first = psq
        if t == T - 1:
            psq_last = psq

    xs = jnp.stack(xs_cols, axis=1)
    s0 = jnp.sum(psq_first[:, 0, :2], axis=0)
    sl = jnp.sum(psq_last[:, 0, :2], axis=0)
    init_norm = jnp.mean(jnp.sqrt(s0))
    last_norm = jnp.mean(jnp.sqrt(sl))
    return xs, init_norm, last_norm
